# baseline (speedup 1.0000x reference)
import functools

import jax
import jax.numpy as jnp
from jax import lax
from jax.experimental import pallas as pl
from jax.experimental.pallas import tpu as pltpu
from jax.experimental.pallas import tpu_sc as plsc

TOP_K = 8


def _router_block_kernel(x_ref, w_ref, scores_ref, sel_ref, rank_ref,
                         hist_ref, offs_ref, acc_ref):
    T = x_ref.shape[0]
    E = w_ref.shape[0]

    @pl.when(pl.program_id(0) == 0)
    def _init():
        acc_ref[...] = jnp.zeros_like(acc_ref)

    scores = lax.dot_general(x_ref[...], w_ref[...], (((1,), (1,)), ((), ())),
                             preferred_element_type=jnp.float32)
    m = jnp.max(scores, axis=1, keepdims=True)
    ex = jnp.exp(scores - m)
    p = ex / jnp.sum(ex, axis=1, keepdims=True)

    lanes = lax.broadcasted_iota(jnp.int32, (T, E), 1)
    s = p
    ohs = []
    cols_score = []
    cols_sel = []
    tokhot = jnp.zeros((T, E), jnp.float32)
    for _ in range(TOP_K):
        mx = jnp.max(s, axis=1, keepdims=True)
        idx = jnp.min(jnp.where(s == mx, lanes, E), axis=1, keepdims=True)
        oh = lanes == idx
        ohf = oh.astype(jnp.float32)
        ohs.append(ohf)
        cols_score.append(mx)
        cols_sel.append(idx)
        tokhot = tokhot + ohf
        s = jnp.where(oh, -jnp.inf, s)

    ri = lax.broadcasted_iota(jnp.int32, (T, T), 0)
    ci = lax.broadcasted_iota(jnp.int32, (T, T), 1)
    ltri = (ri > ci).astype(jnp.float32)
    excl = lax.dot_general(ltri, tokhot, (((1,), (0,)), ((), ())),
                           preferred_element_type=jnp.float32)
    base = excl + acc_ref[0:1, 0:E]
    cols_rank = [jnp.sum(base * ohs[k], axis=1, keepdims=True)
                 for k in range(TOP_K)]

    scores_ref[...] = jnp.concatenate(cols_score, axis=1)
    sel_ref[...] = jnp.concatenate(cols_sel, axis=1)
    rank_ref[...] = jnp.concatenate(cols_rank, axis=1).astype(jnp.int32)

    new_acc = acc_ref[0:1, 0:E] + jnp.sum(tokhot, axis=0, keepdims=True)
    acc_ref[0:1, 0:E] = new_acc
    hist_ref[...] = jnp.broadcast_to(new_acc, hist_ref.shape).astype(jnp.int32)

    ei = lax.broadcasted_iota(jnp.int32, (E, E), 0)
    ej = lax.broadcasted_iota(jnp.int32, (E, E), 1)
    utri = (ei < ej).astype(jnp.float32)
    offs = lax.dot_general(new_acc, utri, (((1,), (0,)), ((), ())),
                           preferred_element_type=jnp.float32,
                           precision=lax.Precision.HIGHEST)
    offs_ref[...] = jnp.broadcast_to(offs, offs_ref.shape).astype(jnp.int32)


def _router_tc(x, gate_weight, block_t):
    n, d = x.shape
    e = gate_weight.shape[0]
    grid = (n // block_t,)
    return pl.pallas_call(
        _router_block_kernel,
        grid=grid,
        in_specs=[
            pl.BlockSpec((block_t, d), lambda i: (i, 0)),
            pl.BlockSpec((e, d), lambda i: (0, 0)),
        ],
        out_specs=[
            pl.BlockSpec((block_t, TOP_K), lambda i: (i, 0)),
            pl.BlockSpec((block_t, TOP_K), lambda i: (i, 0)),
            pl.BlockSpec((block_t, TOP_K), lambda i: (i, 0)),
            pl.BlockSpec((8, e), lambda i: (0, 0)),
            pl.BlockSpec((8, e), lambda i: (0, 0)),
        ],
        out_shape=[
            jax.ShapeDtypeStruct((n, TOP_K), jnp.float32),
            jax.ShapeDtypeStruct((n, TOP_K), jnp.int32),
            jax.ShapeDtypeStruct((n, TOP_K), jnp.int32),
            jax.ShapeDtypeStruct((8, e), jnp.int32),
            jax.ShapeDtypeStruct((8, e), jnp.int32),
        ],
        scratch_shapes=[pltpu.VMEM((8, 128), jnp.float32)],
    )(x, gate_weight)


def _dispatch_sc(scores_f, sel_f, rank_f, offs):
    total = scores_f.shape[0]
    e = offs.shape[0]
    info = plsc.get_sparse_core_info()
    nw = info.num_cores * info.num_subcores
    ch = total // nw
    nrow = ch // 128
    mesh = plsc.VectorSubcoreMesh(core_axis_name="c", subcore_axis_name="s")

    @functools.partial(
        pl.kernel,
        mesh=mesh,
        compiler_params=pltpu.CompilerParams(needs_layout_passes=False),
        out_type=[
            jax.ShapeDtypeStruct((total,), jnp.float32),
            jax.ShapeDtypeStruct((total,), jnp.int32),
        ],
        scratch_types=[
            pltpu.VMEM((e,), jnp.int32),
            pltpu.VMEM((ch,), jnp.int32),
            pltpu.VMEM((ch,), jnp.int32),
            pltpu.VMEM((ch,), jnp.float32),
            pltpu.VMEM((ch,), jnp.int32),
            pltpu.VMEM((nrow, 128), jnp.int32),
            pltpu.SemaphoreType.DMA,
        ],
    )
    def dispatch(scores_hbm, sel_hbm, rank_hbm, offs_hbm,
                 out_s_hbm, out_t_hbm,
                 offs_v, sel_v, rank_v, sc_v, tok_v, pos_v, sem):
        wid = lax.axis_index("s") * info.num_cores + lax.axis_index("c")
        chunk_base = wid * ch

        pltpu.sync_copy(offs_hbm, offs_v)
        pltpu.sync_copy(sel_hbm.at[pl.ds(chunk_base, ch)], sel_v)
        pltpu.sync_copy(rank_hbm.at[pl.ds(chunk_base, ch)], rank_v)
        pltpu.sync_copy(scores_hbm.at[pl.ds(chunk_base, ch)], sc_v)

        lane = lax.iota(jnp.int32, 16)

        def row(c, _):
            for u in range(8):
                start = (c * 8 + u) * 16
                k16 = sel_v[pl.ds(start, 16)]
                r16 = rank_v[pl.ds(start, 16)]
                off16 = plsc.load_gather(offs_v, [k16])
                pos_v[c, pl.ds(u * 16, 16)] = off16 + r16
                flat = chunk_base + start + lane
                tok_v[pl.ds(start, 16)] = flat // TOP_K
            return 0

        lax.fori_loop(0, nrow, row, 0)

        waits = []
        for c in range(nrow):
            waits.append(pltpu.async_copy(
                sc_v.at[pl.ds(c * 128, 128)], out_s_hbm.at[pos_v.at[c]], sem))
        for c in range(nrow):
            waits.append(pltpu.async_copy(
                tok_v.at[pl.ds(c * 128, 128)], out_t_hbm.at[pos_v.at[c]], sem))
        for w in waits:
            w.wait()

    return dispatch(scores_f, sel_f, rank_f, offs)


def kernel(x, gate_weight):
    n, _ = x.shape
    scores, sel, rank, hist8, offs8 = _router_tc(x, gate_weight, block_t=256)
    hist = hist8[0]
    out_s, out_t = _dispatch_sc(scores.reshape(-1), sel.reshape(-1),
                                rank.reshape(-1), offs8[0])
    return out_s, out_t, hist
